# baseline (speedup 1.0000x reference)
import functools

import jax
import jax.numpy as jnp
from jax import lax
from jax.experimental import pallas as pl
from jax.experimental.pallas import tpu as pltpu
from jax.experimental.pallas import tpu_sc as plsc

BATCH = 16384
EMB = 128

_INFO = plsc.get_sparse_core_info()
_NC, _NS = _INFO.num_cores, _INFO.num_subcores
_NW = _NC * _NS
_BPW = BATCH // _NW
_CH = 128
_NCH = _BPW // _CH

_mesh = plsc.VectorSubcoreMesh(core_axis_name="c", subcore_axis_name="s")


@functools.partial(
    pl.kernel,
    mesh=_mesh,
    out_type=[jax.ShapeDtypeStruct((BATCH, EMB), jnp.float32)] * 4,
    scratch_types=[
        pltpu.VMEM((_NCH, _CH), jnp.int32),
        pltpu.VMEM((_NCH, _CH), jnp.int32),
        pltpu.VMEM((4, _CH, EMB), jnp.float32),
        [pltpu.SemaphoreType.DMA] * 4,
        [pltpu.SemaphoreType.DMA] * 4,
    ],
)
def _gather4(uidx_hbm, midx_hbm, ug_t, mg_t, um_t, mm_t,
             ug_o, mg_o, um_o, mm_o,
             uvec, mvec, ring, gsems, ssems):
    wid = lax.axis_index("s") * _NC + lax.axis_index("c")
    base = wid * _BPW
    for c in range(_NCH):
        pltpu.sync_copy(uidx_hbm.at[pl.ds(base + c * _CH, _CH)], uvec.at[c])
        pltpu.sync_copy(midx_hbm.at[pl.ds(base + c * _CH, _CH)], mvec.at[c])

    jobs = []
    for tab, ivec, out in ((ug_t, uvec, ug_o), (mg_t, mvec, mg_o),
                           (um_t, uvec, um_o), (mm_t, mvec, mm_o)):
        for c in range(_NCH):
            jobs.append((tab, ivec, out, c))

    def gather(j):
        tab, ivec, _, c = jobs[j]
        return pltpu.make_async_copy(tab.at[ivec.at[c]], ring.at[j % 4],
                                     gsems[j % 4])

    def store(j):
        _, _, out, c = jobs[j]
        return pltpu.make_async_copy(ring.at[j % 4],
                                     out.at[pl.ds(base + c * _CH, _CH)],
                                     ssems[j % 4])

    n = len(jobs)
    for j in range(n):
        if j >= 4:
            store(j - 4).wait()
        gather(j).start()
        if j >= 1:
            gather(j - 1).wait()
            store(j - 1).start()
    gather(n - 1).wait()
    store(n - 1).start()
    for j in range(n - 4, n):
        store(j).wait()


_BM = 2048


def _tc_body(ug, mg, um, mm, gmfw, w0a, w0b, b0, w1, b1, w2, b2, w3, b3,
             fmw, cconst, out_ref):
    g = jnp.sum(ug[...] * mg[...] * gmfw[...], axis=1)
    h = jnp.maximum(
        jnp.dot(um[...], w0a[...], preferred_element_type=jnp.float32)
        + jnp.dot(mm[...], w0b[...], preferred_element_type=jnp.float32)
        + b0[...], 0.0)
    h = jnp.maximum(jnp.dot(h, w1[...], preferred_element_type=jnp.float32) + b1[...], 0.0)
    h = jnp.maximum(jnp.dot(h, w2[...], preferred_element_type=jnp.float32) + b2[...], 0.0)
    h = jnp.maximum(jnp.dot(h, w3[...], preferred_element_type=jnp.float32) + b3[...], 0.0)
    m = jnp.sum(h * fmw[...], axis=1)
    out_ref[...] = g + m + cconst[0, 0]


def _full(shape):
    return pl.BlockSpec(shape, lambda i: (0, 0))


_tc_call = pl.pallas_call(
    _tc_body,
    grid=(BATCH // _BM,),
    in_specs=[
        pl.BlockSpec((_BM, EMB), lambda i: (i, 0)),
        pl.BlockSpec((_BM, EMB), lambda i: (i, 0)),
        pl.BlockSpec((_BM, EMB), lambda i: (i, 0)),
        pl.BlockSpec((_BM, EMB), lambda i: (i, 0)),
        _full((1, EMB)),
        _full((EMB, 64)),
        _full((EMB, 64)),
        _full((1, 64)),
        _full((64, 32)),
        _full((1, 32)),
        _full((32, 16)),
        _full((1, 16)),
        _full((16, 8)),
        _full((1, 8)),
        _full((1, 8)),
        _full((1, 1)),
    ],
    out_specs=pl.BlockSpec((_BM,), lambda i: (i,)),
    out_shape=jax.ShapeDtypeStruct((BATCH,), jnp.float32),
)


def kernel(X, user_emb_gmf, movie_emb_gmf, user_emb_mlp, movie_emb_mlp,
           gmf_w, gmf_b, final_mlp_w, final_mlp_b, final_w, final_b,
           mlp_w0, mlp_b0, mlp_w1, mlp_b1, mlp_w2, mlp_b2, mlp_w3, mlp_b3):
    user = X[:, 0]
    movie = X[:, 1]
    ug, mg, um, mm = _gather4(user, movie, user_emb_gmf, movie_emb_gmf,
                              user_emb_mlp, movie_emb_mlp)
    fw0 = final_w[0, 0]
    fw1 = final_w[1, 0]
    gmfw = (gmf_w[:, 0] * fw0).reshape(1, EMB)
    fmw = (final_mlp_w[:, 0] * fw1).reshape(1, 8)
    cconst = (final_b[0] + fw0 * gmf_b[0] + fw1 * final_mlp_b[0]).reshape(1, 1)
    out = _tc_call(ug, mg, um, mm, gmfw,
                   mlp_w0[:EMB], mlp_w0[EMB:], mlp_b0.reshape(1, -1),
                   mlp_w1, mlp_b1.reshape(1, -1),
                   mlp_w2, mlp_b2.reshape(1, -1),
                   mlp_w3, mlp_b3.reshape(1, -1),
                   fmw, cconst)
    return out.reshape(BATCH, 1)


# speedup vs baseline: 1.0404x; 1.0404x over previous
import functools

import jax
import jax.numpy as jnp
from jax import lax
from jax.experimental import pallas as pl
from jax.experimental.pallas import tpu as pltpu
from jax.experimental.pallas import tpu_sc as plsc

BATCH = 16384
EMB = 128

_INFO = plsc.get_sparse_core_info()
_NC, _NS = _INFO.num_cores, _INFO.num_subcores
_NW = _NC * _NS
_BPW = BATCH // _NW
_CH = 128
_NCH = _BPW // _CH

_mesh = plsc.VectorSubcoreMesh(core_axis_name="c", subcore_axis_name="s")


@functools.partial(
    pl.kernel,
    mesh=_mesh,
    out_type=[
        jax.ShapeDtypeStruct((BATCH, EMB), jnp.float32),
        jax.ShapeDtypeStruct((BATCH, EMB), jnp.float32),
        jax.ShapeDtypeStruct((BATCH, 16), jnp.float32),
    ],
    scratch_types=[
        pltpu.VMEM((_NCH, _CH), jnp.int32),
        pltpu.VMEM((_NCH, _CH), jnp.int32),
        pltpu.VMEM((EMB,), jnp.float32),
        pltpu.VMEM((4, _CH, EMB), jnp.float32),
        pltpu.VMEM((_CH, 16), jnp.float32),
        [pltpu.SemaphoreType.DMA] * 4,
        [pltpu.SemaphoreType.DMA] * 2,
    ],
)
def _sc_gather(uidx_hbm, midx_hbm, ug_t, mg_t, um_t, mm_t, gmfw_hbm,
               um_o, mm_o, gmf_o,
               uvec, mvec, wvec, ring, gmfv, gsems, ssems):
    wid = lax.axis_index("s") * _NC + lax.axis_index("c")
    base = wid * _BPW
    pltpu.sync_copy(gmfw_hbm, wvec)
    for c in range(_NCH):
        pltpu.sync_copy(uidx_hbm.at[pl.ds(base + c * _CH, _CH)], uvec.at[c])
        pltpu.sync_copy(midx_hbm.at[pl.ds(base + c * _CH, _CH)], mvec.at[c])

    def gather(tab, ivec, c, s):
        return pltpu.make_async_copy(tab.at[ivec.at[c]], ring.at[s], gsems[s])

    def store(out, c, s, ss):
        return pltpu.make_async_copy(ring.at[s],
                                     out.at[pl.ds(base + c * _CH, _CH)],
                                     ssems[ss])

    def dot_chunk(c):
        ugb = ring.at[2]
        mgb = ring.at[3]

        def group(g, _):
            for r in range(16):
                row = g * 16 + r
                acc = (ugb[row, pl.ds(0, 16)] * mgb[row, pl.ds(0, 16)]
                       * wvec[pl.ds(0, 16)])
                for k in range(1, EMB // 16):
                    acc = acc + (ugb[row, pl.ds(16 * k, 16)]
                                 * mgb[row, pl.ds(16 * k, 16)]
                                 * wvec[pl.ds(16 * k, 16)])
                gmfv[g * 16 + r] = acc
            return 0

        lax.fori_loop(0, _CH // 16, group, 0)

    for c in range(_NCH):
        if c > 0:
            store(um_o, c - 1, 0, 0).wait()
        gather(um_t, uvec, c, 0).start()
        if c > 0:
            store(mm_o, c - 1, 1, 1).wait()
        gather(mm_t, mvec, c, 1).start()
        gather(ug_t, uvec, c, 2).start()
        gather(mg_t, mvec, c, 3).start()
        gather(um_t, uvec, c, 0).wait()
        store(um_o, c, 0, 0).start()
        gather(mm_t, mvec, c, 1).wait()
        store(mm_o, c, 1, 1).start()
        gather(ug_t, uvec, c, 2).wait()
        gather(mg_t, mvec, c, 3).wait()
        dot_chunk(c)
        pltpu.sync_copy(gmfv, gmf_o.at[pl.ds(base + c * _CH, _CH)])
    store(um_o, _NCH - 1, 0, 0).wait()
    store(mm_o, _NCH - 1, 1, 1).wait()


_BM = 2048


def _tc_body(um, mm, gp, w0a, w0b, b0, w1, b1, w2, b2, w3, b3, fmw, cconst,
             out_ref):
    g = jnp.sum(gp[...], axis=1, keepdims=True)
    h = jnp.maximum(
        jnp.dot(um[...], w0a[...], preferred_element_type=jnp.float32)
        + jnp.dot(mm[...], w0b[...], preferred_element_type=jnp.float32)
        + b0[...], 0.0)
    h = jnp.maximum(jnp.dot(h, w1[...], preferred_element_type=jnp.float32) + b1[...], 0.0)
    h = jnp.maximum(jnp.dot(h, w2[...], preferred_element_type=jnp.float32) + b2[...], 0.0)
    h = jnp.maximum(jnp.dot(h, w3[...], preferred_element_type=jnp.float32) + b3[...], 0.0)
    out_ref[...] = (jnp.dot(h, fmw[...], preferred_element_type=jnp.float32)
                    + g + cconst[...])


def _full(shape):
    return pl.BlockSpec(shape, lambda i: (0, 0))


_tc_call = pl.pallas_call(
    _tc_body,
    grid=(BATCH // _BM,),
    in_specs=[
        pl.BlockSpec((_BM, EMB), lambda i: (i, 0)),
        pl.BlockSpec((_BM, EMB), lambda i: (i, 0)),
        pl.BlockSpec((_BM, 16), lambda i: (i, 0)),
        _full((EMB, 64)),
        _full((EMB, 64)),
        _full((1, 64)),
        _full((64, 32)),
        _full((1, 32)),
        _full((32, 16)),
        _full((1, 16)),
        _full((16, 8)),
        _full((1, 8)),
        _full((8, 1)),
        _full((1, 1)),
    ],
    out_specs=pl.BlockSpec((_BM, 1), lambda i: (i, 0)),
    out_shape=jax.ShapeDtypeStruct((BATCH, 1), jnp.float32),
)


def kernel(X, user_emb_gmf, movie_emb_gmf, user_emb_mlp, movie_emb_mlp,
           gmf_w, gmf_b, final_mlp_w, final_mlp_b, final_w, final_b,
           mlp_w0, mlp_b0, mlp_w1, mlp_b1, mlp_w2, mlp_b2, mlp_w3, mlp_b3):
    user = X[:, 0]
    movie = X[:, 1]
    fw0 = final_w[0, 0]
    fw1 = final_w[1, 0]
    gmfw = gmf_w[:, 0] * fw0
    fmw = final_mlp_w * fw1
    cconst = (final_b[0] + fw0 * gmf_b[0] + fw1 * final_mlp_b[0]).reshape(1, 1)
    um, mm, gmf = _sc_gather(user, movie, user_emb_gmf, movie_emb_gmf,
                             user_emb_mlp, movie_emb_mlp, gmfw)
    return _tc_call(um, mm, gmf,
                    mlp_w0[:EMB], mlp_w0[EMB:], mlp_b0.reshape(1, -1),
                    mlp_w1, mlp_b1.reshape(1, -1),
                    mlp_w2, mlp_b2.reshape(1, -1),
                    mlp_w3, mlp_b3.reshape(1, -1),
                    fmw, cconst)


# speedup vs baseline: 1.2590x; 1.2101x over previous
import functools

import jax
import jax.numpy as jnp
from jax import lax
from jax.experimental import pallas as pl
from jax.experimental.pallas import tpu as pltpu
from jax.experimental.pallas import tpu_sc as plsc

BATCH = 16384
EMB = 128
_HALF = BATCH // 2

_INFO = plsc.get_sparse_core_info()
_NC, _NS = _INFO.num_cores, _INFO.num_subcores
_NW = _NC * _NS
_BPW = _HALF // _NW
_CH = 128
_NCH = _BPW // _CH

_mesh = plsc.VectorSubcoreMesh(core_axis_name="c", subcore_axis_name="s")


@functools.partial(
    pl.kernel,
    mesh=_mesh,
    out_type=[jax.ShapeDtypeStruct((_HALF, EMB), jnp.float32)] * 4,
    scratch_types=[
        pltpu.VMEM((_NCH, _CH), jnp.int32),
        pltpu.VMEM((_NCH, _CH), jnp.int32),
        pltpu.VMEM((4, _CH, EMB), jnp.float32),
        [pltpu.SemaphoreType.DMA] * 4,
        [pltpu.SemaphoreType.DMA] * 4,
    ],
)
def _sc_gather4(uidx_hbm, midx_hbm, ug_t, mg_t, um_t, mm_t,
                ug_o, mg_o, um_o, mm_o,
                uvec, mvec, ring, gsems, ssems):
    wid = lax.axis_index("s") * _NC + lax.axis_index("c")
    base = wid * _BPW
    for c in range(_NCH):
        pltpu.sync_copy(uidx_hbm.at[pl.ds(base + c * _CH, _CH)], uvec.at[c])
        pltpu.sync_copy(midx_hbm.at[pl.ds(base + c * _CH, _CH)], mvec.at[c])

    jobs = []
    for tab, ivec, out in ((ug_t, uvec, ug_o), (mg_t, mvec, mg_o),
                           (um_t, uvec, um_o), (mm_t, mvec, mm_o)):
        for c in range(_NCH):
            jobs.append((tab, ivec, out, c))

    def gather(j):
        tab, ivec, _, c = jobs[j]
        return pltpu.make_async_copy(tab.at[ivec.at[c]], ring.at[j % 4],
                                     gsems[j % 4])

    def store(j):
        _, _, out, c = jobs[j]
        return pltpu.make_async_copy(ring.at[j % 4],
                                     out.at[pl.ds(base + c * _CH, _CH)],
                                     ssems[j % 4])

    n = len(jobs)
    for j in range(n):
        if j >= 4:
            store(j - 4).wait()
        gather(j).start()
        if j >= 1:
            gather(j - 1).wait()
            store(j - 1).start()
    gather(n - 1).wait()
    store(n - 1).start()
    for j in range(max(0, n - 4), n):
        store(j).wait()


_BM = 2048


def _nt(a, b):
    return lax.dot_general(a, b, (((1,), (1,)), ((), ())),
                           preferred_element_type=jnp.float32)


def _tc_body(ug, mg, um, mm, gmfwr, w0at, w0bt, b0c, w1t, b1c, w2t, b2c,
             w3t, b3c, fmwt, cconst, out_ref):
    h = jnp.maximum(_nt(w0at[...], um[...]) + _nt(w0bt[...], mm[...])
                    + b0c[...], 0.0)
    h = jnp.maximum(jnp.dot(w1t[...], h, preferred_element_type=jnp.float32)
                    + b1c[...], 0.0)
    h = jnp.maximum(jnp.dot(w2t[...], h, preferred_element_type=jnp.float32)
                    + b2c[...], 0.0)
    h = jnp.maximum(jnp.dot(w3t[...], h, preferred_element_type=jnp.float32)
                    + b3c[...], 0.0)
    m = jnp.dot(fmwt[...], h, preferred_element_type=jnp.float32)
    g = _nt(gmfwr[...], ug[...] * mg[...])
    out_ref[...] = (m + g + cconst[...])[0]


def _full(shape):
    return pl.BlockSpec(shape, lambda i: (0, 0))


def _row(shape):
    return pl.BlockSpec(shape, lambda i: (i, 0))


_tc_call = pl.pallas_call(
    _tc_body,
    grid=(_HALF // _BM,),
    in_specs=[
        _row((_BM, EMB)),
        _row((_BM, EMB)),
        _row((_BM, EMB)),
        _row((_BM, EMB)),
        _full((1, EMB)),
        _full((64, EMB)),
        _full((64, EMB)),
        _full((64, 1)),
        _full((32, 64)),
        _full((32, 1)),
        _full((16, 32)),
        _full((16, 1)),
        _full((8, 16)),
        _full((8, 1)),
        _full((1, 8)),
        _full((1, 1)),
    ],
    out_specs=pl.BlockSpec((_BM,), lambda i: (i,)),
    out_shape=jax.ShapeDtypeStruct((_HALF,), jnp.float32),
)


def kernel(X, user_emb_gmf, movie_emb_gmf, user_emb_mlp, movie_emb_mlp,
           gmf_w, gmf_b, final_mlp_w, final_mlp_b, final_w, final_b,
           mlp_w0, mlp_b0, mlp_w1, mlp_b1, mlp_w2, mlp_b2, mlp_w3, mlp_b3):
    user = X[:, 0]
    movie = X[:, 1]
    fw0 = final_w[0, 0]
    fw1 = final_w[1, 0]
    gmfwr = (gmf_w[:, 0] * fw0).reshape(1, EMB)
    fmwt = (final_mlp_w[:, 0] * fw1).reshape(1, 8)
    cconst = (final_b[0] + fw0 * gmf_b[0] + fw1 * final_mlp_b[0]).reshape(1, 1)
    wts = (gmfwr, mlp_w0[:EMB].T, mlp_w0[EMB:].T, mlp_b0.reshape(-1, 1),
           mlp_w1.T, mlp_b1.reshape(-1, 1), mlp_w2.T, mlp_b2.reshape(-1, 1),
           mlp_w3.T, mlp_b3.reshape(-1, 1), fmwt, cconst)
    outs = []
    for h in range(2):
        sl = slice(h * _HALF, (h + 1) * _HALF)
        ug, mg, um, mm = _sc_gather4(user[sl], movie[sl],
                                     user_emb_gmf, movie_emb_gmf,
                                     user_emb_mlp, movie_emb_mlp)
        outs.append(_tc_call(ug, mg, um, mm, *wts))
    return jnp.concatenate(outs).reshape(BATCH, 1)
